# baseline (speedup 1.0000x reference)
import functools

import jax
import jax.numpy as jnp
from jax import lax
from jax.experimental import pallas as pl
from jax.experimental.pallas import tpu as pltpu
from jax.experimental.pallas import tpu_sc as plsc

NUM_ROWS = 1000000
DIM = 64
N_EX = 16384
N_TOK = 200

N_WORKERS = 32
EX_PER_W = N_EX // N_WORKERS
EX_CHUNK = 128
N_BC = EX_PER_W // EX_CHUNK
T_CHUNK = 2
N_TC = N_TOK // T_CHUNK
ROWS_PER_BLK = EX_CHUNK * T_CHUNK


def _make_kernel():
    mesh = plsc.VectorSubcoreMesh(core_axis_name="c", subcore_axis_name="s")
    nc = plsc.get_sparse_core_info().num_cores

    @functools.partial(
        pl.kernel,
        mesh=mesh,
        out_type=jax.ShapeDtypeStruct((N_TOK, DIM, N_EX), jnp.float32),
        scratch_types=[
            pltpu.VMEM((EX_CHUNK * N_TOK,), jnp.int32),
            pltpu.VMEM((ROWS_PER_BLK,), jnp.int32),
            pltpu.VMEM((ROWS_PER_BLK,), jnp.int32),
            pltpu.VMEM((ROWS_PER_BLK, 2 * DIM), jnp.float32),
            pltpu.VMEM((T_CHUNK, DIM, EX_CHUNK), jnp.float32),
            pltpu.SemaphoreType.DMA,
            pltpu.SemaphoreType.DMA,
        ],
        compiler_params=pltpu.CompilerParams(needs_layout_passes=False),
    )
    def emb_kernel(idx_hbm, table_hbm, out_hbm, idx_v, lv, pv, gv, tv, sg, so):
        wid = lax.axis_index("s") * nc + lax.axis_index("c")
        iota = lax.iota(jnp.int32, 16)
        iota_tok = iota * N_TOK

        def bc_body(bc, _):
            ex0 = wid * EX_PER_W + bc * EX_CHUNK
            pltpu.sync_copy(
                idx_hbm.at[pl.ds(pl.multiple_of(ex0 * N_TOK, 8), EX_CHUNK * N_TOK)],
                idx_v,
            )

            def tc_body(tc, _):
                t0 = tc * T_CHUNK
                for t_loc in range(T_CHUNK):
                    for b16 in range(EX_CHUNK // 16):
                        r0 = t_loc * EX_CHUNK + b16 * 16
                        v = plsc.load_gather(
                            idx_v, [iota_tok + (b16 * 16 * N_TOK + t0 + t_loc)]
                        )
                        lv[pl.ds(r0, 16)] = lax.shift_right_logical(v, 1)
                        pv[pl.ds(r0, 16)] = lax.shift_left((v & 1), 6)
                cp_g = pltpu.make_async_copy(table_hbm.at[lv], gv, sg)
                cp_g.start()
                cp_g.wait()
                for t_loc in range(T_CHUNK):
                    for b16 in range(EX_CHUNK // 16):
                        r0 = t_loc * EX_CHUNK + b16 * 16
                        rowv = iota + r0
                        colv = pv[pl.ds(r0, 16)]
                        for d in range(DIM):
                            tv[t_loc, d, pl.ds(b16 * 16, 16)] = plsc.load_gather(
                                gv, [rowv, colv + d]
                            )
                cp_o = pltpu.make_async_copy(
                    tv,
                    out_hbm.at[
                        pl.ds(t0, T_CHUNK), :, pl.ds(pl.multiple_of(ex0, 128), EX_CHUNK)
                    ],
                    so,
                )
                cp_o.start()
                cp_o.wait()
                return 0

            lax.fori_loop(0, N_TC, tc_body, 0)
            return 0

        lax.fori_loop(0, N_BC, bc_body, 0)

    return emb_kernel


def kernel(indices, weight):
    idx_flat = indices.reshape(-1).astype(jnp.int32)
    table2 = weight.reshape(NUM_ROWS // 2, 2 * DIM)
    out_phys = _make_kernel()(idx_flat, table2)
    return jnp.transpose(out_phys, (2, 0, 1))


# speedup vs baseline: 1.1039x; 1.1039x over previous
import functools

import jax
import jax.numpy as jnp
from jax import lax
from jax.experimental import pallas as pl
from jax.experimental.pallas import tpu as pltpu
from jax.experimental.pallas import tpu_sc as plsc

NUM_ROWS = 1000000
DIM = 64
N_EX = 16384
N_TOK = 200

N_WORKERS = 32
EX_PER_W = N_EX // N_WORKERS
EX_CHUNK = 128
N_BC = EX_PER_W // EX_CHUNK
T_CHUNK = 1
N_TC = N_TOK // T_CHUNK
ROWS_PER_BLK = EX_CHUNK * T_CHUNK


def _make_kernel():
    mesh = plsc.VectorSubcoreMesh(core_axis_name="c", subcore_axis_name="s")
    nc = plsc.get_sparse_core_info().num_cores

    @functools.partial(
        pl.kernel,
        mesh=mesh,
        out_type=jax.ShapeDtypeStruct((N_TOK, DIM, N_EX), jnp.float32),
        scratch_types=[
            pltpu.VMEM((EX_PER_W // N_BC * N_TOK,), jnp.int32),
            pltpu.VMEM((ROWS_PER_BLK,), jnp.int32),
            pltpu.VMEM((ROWS_PER_BLK,), jnp.int32),
            pltpu.VMEM((2, ROWS_PER_BLK), jnp.int32),
            pltpu.VMEM((2, ROWS_PER_BLK, 2 * DIM), jnp.float32),
            pltpu.VMEM((2, T_CHUNK, DIM, EX_CHUNK), jnp.float32),
            pltpu.SemaphoreType.DMA,
            pltpu.SemaphoreType.DMA,
            pltpu.SemaphoreType.DMA,
            pltpu.SemaphoreType.DMA,
        ],
        compiler_params=pltpu.CompilerParams(needs_layout_passes=False),
    )
    def emb_kernel(
        idx_hbm, table_hbm, out_hbm, idx_v, lv0, lv1, pv, gv, tv, sg0, sg1, so0, so1
    ):
        wid = lax.axis_index("s") * nc + lax.axis_index("c")
        iota = lax.iota(jnp.int32, 16)
        iota_tok = iota * N_TOK
        sg = (sg0, sg1)
        so = (so0, so1)
        lv = (lv0, lv1)

        def bc_body(bc, _):
            ex0 = wid * EX_PER_W + bc * EX_CHUNK
            pltpu.sync_copy(
                idx_hbm.at[pl.ds(pl.multiple_of(ex0 * N_TOK, 8), EX_CHUNK * N_TOK)],
                idx_v,
            )

            def prep_and_gather(tc, p):
                t0 = tc * T_CHUNK
                for t_loc in range(T_CHUNK):
                    for b16 in range(EX_CHUNK // 16):
                        r0 = t_loc * EX_CHUNK + b16 * 16
                        v = plsc.load_gather(
                            idx_v, [iota_tok + (b16 * 16 * N_TOK + t0 + t_loc)]
                        )
                        lv[p][pl.ds(r0, 16)] = lax.shift_right_logical(v, 1)
                        pv[p, pl.ds(r0, 16)] = lax.shift_left((v & 1), 6)
                pltpu.make_async_copy(table_hbm.at[lv[p]], gv.at[p], sg[p]).start()

            def transpose(p):
                for t_loc in range(T_CHUNK):
                    rowvs = [iota + (t_loc * EX_CHUNK + b16 * 16) for b16 in range(8)]
                    colvs = [
                        pv[p, pl.ds(t_loc * EX_CHUNK + b16 * 16, 16)]
                        for b16 in range(8)
                    ]
                    for d in range(DIM):
                        for b16 in range(8):
                            tv[p, t_loc, d, pl.ds(b16 * 16, 16)] = plsc.load_gather(
                                gv.at[p], [rowvs[b16], colvs[b16] + d]
                            )

            def store_copy(tc, p):
                t0 = tc * T_CHUNK
                return pltpu.make_async_copy(
                    tv.at[p],
                    out_hbm.at[
                        pl.ds(t0, T_CHUNK),
                        :,
                        pl.ds(pl.multiple_of(ex0, 128), EX_CHUNK),
                    ],
                    so[p],
                )

            def gather_wait(p):
                pltpu.make_async_copy(table_hbm.at[lv[p]], gv.at[p], sg[p]).wait()

            prep_and_gather(0, 0)
            prep_and_gather(1, 1)

            def tc_body(t, _):
                for p in range(2):
                    g = 2 * t + p
                    gather_wait(p)

                    @pl.when(t > 0)
                    def _():
                        store_copy(g, p).wait()

                    transpose(p)
                    store_copy(g, p).start()

                    @pl.when(t < N_TC // 2 - 1)
                    def _():
                        prep_and_gather(g + 2, p)

                return 0

            lax.fori_loop(0, N_TC // 2, tc_body, 0)
            store_copy(N_TC - 2, 0).wait()
            store_copy(N_TC - 1, 1).wait()
            return 0

        lax.fori_loop(0, N_BC, bc_body, 0)

    return emb_kernel


def kernel(indices, weight):
    idx_flat = indices.reshape(-1).astype(jnp.int32)
    table2 = weight.reshape(NUM_ROWS // 2, 2 * DIM)
    out_phys = _make_kernel()(idx_flat, table2)
    return jnp.transpose(out_phys, (2, 0, 1))


# speedup vs baseline: 1.7857x; 1.6176x over previous
import functools

import jax
import jax.numpy as jnp
from jax import lax
from jax.experimental import pallas as pl
from jax.experimental.pallas import tpu as pltpu
from jax.experimental.pallas import tpu_sc as plsc

NUM_ROWS = 1000000
DIM = 64
N_EX = 16384
N_TOK = 200

N_WORKERS = 32
EX_PER_W = N_EX // N_WORKERS
EX_CHUNK = 128
N_BC = EX_PER_W // EX_CHUNK
T_CHUNK = 1
N_TC = N_TOK // T_CHUNK
ROWS_PER_BLK = EX_CHUNK * T_CHUNK


def _make_kernel():
    mesh = plsc.VectorSubcoreMesh(core_axis_name="c", subcore_axis_name="s")
    nc = plsc.get_sparse_core_info().num_cores

    @functools.partial(
        pl.kernel,
        mesh=mesh,
        out_type=jax.ShapeDtypeStruct((N_TOK, DIM, N_EX), jnp.float32),
        scratch_types=[
            pltpu.VMEM((EX_PER_W // N_BC * N_TOK,), jnp.int32),
            pltpu.VMEM((ROWS_PER_BLK,), jnp.int32),
            pltpu.VMEM((ROWS_PER_BLK,), jnp.int32),
            pltpu.VMEM((2, ROWS_PER_BLK + 16), jnp.int32),
            pltpu.VMEM((2, ROWS_PER_BLK, 2 * DIM), jnp.float32),
            pltpu.VMEM((T_CHUNK, DIM, EX_CHUNK + 5), jnp.float32),
            pltpu.VMEM((T_CHUNK, DIM, EX_CHUNK + 5), jnp.float32),
            pltpu.SemaphoreType.DMA,
            pltpu.SemaphoreType.DMA,
            pltpu.SemaphoreType.DMA,
            pltpu.SemaphoreType.DMA,
        ],
        compiler_params=pltpu.CompilerParams(needs_layout_passes=False),
    )
    def emb_kernel(
        idx_hbm, table_hbm, out_hbm, idx_v, lv0, lv1, pv, gv, tv0, tv1, sg0, sg1, so0, so1
    ):
        wid = lax.axis_index("s") * nc + lax.axis_index("c")
        iota = lax.iota(jnp.int32, 16)
        iota_tok = iota * N_TOK
        sg = (sg0, sg1)
        so = (so0, so1)
        lv = (lv0, lv1)
        tv = (tv0, tv1)
        zero16 = iota * 0

        def bc_body(bc, _):
            ex0 = wid * EX_PER_W + bc * EX_CHUNK
            pltpu.sync_copy(
                idx_hbm.at[pl.ds(pl.multiple_of(ex0 * N_TOK, 8), EX_CHUNK * N_TOK)],
                idx_v,
            )

            def prep_and_gather(tc, p):
                t0 = tc * T_CHUNK
                for t_loc in range(T_CHUNK):
                    for b16 in range(EX_CHUNK // 16):
                        r0 = t_loc * EX_CHUNK + b16 * 16
                        v = plsc.load_gather(
                            idx_v, [iota_tok + (b16 * 16 * N_TOK + t0 + t_loc)]
                        )
                        lv[p][pl.ds(r0, 16)] = lax.shift_right_logical(v, 1)
                        pv[p, pl.ds(r0, 16)] = lax.shift_left((v & 1), 6)
                pltpu.make_async_copy(table_hbm.at[lv[p]], gv.at[p], sg[p]).start()

            def transpose(p):
                @plsc.parallel_loop(0, ROWS_PER_BLK, unroll=8)
                def _(r):
                    c0 = pv[p, pl.ds(r, 16)][0]
                    rv = zero16 + r
                    for q in range(DIM // 16):
                        plsc.store_scatter(
                            tv[p],
                            [zero16, iota + q * 16, rv],
                            gv[p, r, pl.ds(c0 + q * 16, 16)],
                        )

            def store_copy(tc, p):
                t0 = tc * T_CHUNK
                return pltpu.make_async_copy(
                    tv[p].at[:, :, pl.ds(0, EX_CHUNK)],
                    out_hbm.at[
                        pl.ds(t0, T_CHUNK),
                        :,
                        pl.ds(pl.multiple_of(ex0, 128), EX_CHUNK),
                    ],
                    so[p],
                )

            def gather_wait(p):
                pltpu.make_async_copy(table_hbm.at[lv[p]], gv.at[p], sg[p]).wait()

            prep_and_gather(0, 0)
            prep_and_gather(1, 1)

            def tc_body(t, _):
                for p in range(2):
                    g = 2 * t + p
                    gather_wait(p)

                    @pl.when(t > 0)
                    def _():
                        store_copy(g, p).wait()

                    transpose(p)
                    store_copy(g, p).start()

                    @pl.when(t < N_TC // 2 - 1)
                    def _():
                        prep_and_gather(g + 2, p)

                return 0

            lax.fori_loop(0, N_TC // 2, tc_body, 0)
            store_copy(N_TC - 2, 0).wait()
            store_copy(N_TC - 1, 1).wait()
            return 0

        lax.fori_loop(0, N_BC, bc_body, 0)

    return emb_kernel


def kernel(indices, weight):
    idx_flat = indices.reshape(-1).astype(jnp.int32)
    table2 = weight.reshape(NUM_ROWS // 2, 2 * DIM)
    out_phys = _make_kernel()(idx_flat, table2)
    return jnp.transpose(out_phys, (2, 0, 1))


# speedup vs baseline: 1.9511x; 1.0926x over previous
import functools

import jax
import jax.numpy as jnp
from jax import lax
from jax.experimental import pallas as pl
from jax.experimental.pallas import tpu as pltpu
from jax.experimental.pallas import tpu_sc as plsc

NUM_ROWS = 1000000
DIM = 64
N_EX = 16384
N_TOK = 200

N_WORKERS = 32
EX_PER_W = N_EX // N_WORKERS
EX_CHUNK = 128
N_BC = EX_PER_W // EX_CHUNK
T_CHUNK = 1
N_TC = N_TOK // T_CHUNK
ROWS_PER_BLK = EX_CHUNK * T_CHUNK


def _make_kernel():
    mesh = plsc.VectorSubcoreMesh(core_axis_name="c", subcore_axis_name="s")
    nc = plsc.get_sparse_core_info().num_cores

    @functools.partial(
        pl.kernel,
        mesh=mesh,
        out_type=jax.ShapeDtypeStruct((N_TOK, DIM, N_EX), jnp.float32),
        scratch_types=[
            pltpu.VMEM((EX_PER_W // N_BC * N_TOK,), jnp.int32),
            pltpu.VMEM((ROWS_PER_BLK,), jnp.int32),
            pltpu.VMEM((ROWS_PER_BLK,), jnp.int32),
            pltpu.VMEM((2, ROWS_PER_BLK, 2 * DIM), jnp.float32),
            pltpu.VMEM((T_CHUNK, DIM, EX_CHUNK + 5), jnp.float32),
            pltpu.VMEM((T_CHUNK, DIM, EX_CHUNK + 5), jnp.float32),
            pltpu.SemaphoreType.DMA,
            pltpu.SemaphoreType.DMA,
            pltpu.SemaphoreType.DMA,
            pltpu.SemaphoreType.DMA,
        ],
        compiler_params=pltpu.CompilerParams(needs_layout_passes=False),
    )
    def emb_kernel(
        idx_hbm, table_hbm, out_hbm, idx_v, lv0, lv1, gv, tv0, tv1, sg0, sg1, so0, so1
    ):
        wid = lax.axis_index("s") * nc + lax.axis_index("c")
        iota = lax.iota(jnp.int32, 16)
        iota_tok = iota * N_TOK
        sg = (sg0, sg1)
        so = (so0, so1)
        lv = (lv0, lv1)
        tv = (tv0, tv1)
        zero16 = iota * 0

        def bc_body(bc, _):
            ex0 = wid * EX_PER_W + bc * EX_CHUNK
            pltpu.sync_copy(
                idx_hbm.at[pl.ds(pl.multiple_of(ex0 * N_TOK, 8), EX_CHUNK * N_TOK)],
                idx_v,
            )

            def prep_and_gather(tc, p):
                t0 = tc * T_CHUNK
                for t_loc in range(T_CHUNK):
                    for b16 in range(EX_CHUNK // 16):
                        r0 = t_loc * EX_CHUNK + b16 * 16
                        v = plsc.load_gather(
                            idx_v, [iota_tok + (b16 * 16 * N_TOK + t0 + t_loc)]
                        )
                        lv[p][pl.ds(r0, 16)] = v
                pltpu.make_async_copy(table_hbm.at[lv[p]], gv.at[p], sg[p]).start()

            def transpose(p):
                @plsc.parallel_loop(0, ROWS_PER_BLK, unroll=8)
                def _(r):
                    rv = zero16 + r
                    for q in range(DIM // 16):
                        plsc.store_scatter(
                            tv[p],
                            [zero16, iota + q * 16, rv],
                            gv[p, r, pl.ds(q * 16, 16)],
                        )

            def store_copy(tc, p):
                t0 = tc * T_CHUNK
                return pltpu.make_async_copy(
                    tv[p].at[:, :, pl.ds(0, EX_CHUNK)],
                    out_hbm.at[
                        pl.ds(t0, T_CHUNK),
                        :,
                        pl.ds(pl.multiple_of(ex0, 128), EX_CHUNK),
                    ],
                    so[p],
                )

            def gather_wait(p):
                pltpu.make_async_copy(table_hbm.at[lv[p]], gv.at[p], sg[p]).wait()

            prep_and_gather(0, 0)
            prep_and_gather(1, 1)

            def tc_body(t, _):
                for p in range(2):
                    g = 2 * t + p
                    gather_wait(p)

                    @pl.when(t > 0)
                    def _():
                        store_copy(g, p).wait()

                    transpose(p)
                    store_copy(g, p).start()

                    @pl.when(t < N_TC // 2 - 1)
                    def _():
                        prep_and_gather(g + 2, p)

                return 0

            lax.fori_loop(0, N_TC // 2, tc_body, 0)
            store_copy(N_TC - 2, 0).wait()
            store_copy(N_TC - 1, 1).wait()
            return 0

        lax.fori_loop(0, N_BC, bc_body, 0)

    return emb_kernel


def kernel(indices, weight):
    idx_flat = indices.reshape(-1).astype(jnp.int32)
    table_pad = jnp.pad(weight, ((0, 0), (0, DIM)))
    out_phys = _make_kernel()(idx_flat, table_pad)
    return jnp.transpose(out_phys, (2, 0, 1))
